# baseline (speedup 1.0000x reference)
import functools

import jax
import jax.numpy as jnp
from jax import lax
from jax.experimental import pallas as pl
from jax.experimental.pallas import tpu as pltpu
from jax.experimental.pallas import tpu_sc as plsc

D = 32
L = 16

_SHUF_DNUMS = lax.GatherDimensionNumbers(
    offset_dims=(), collapsed_slice_dims=(0,), start_index_map=(0,))


def _shuffle(v, idx):
  return lax.gather(v, idx[:, None], _SHUF_DNUMS, (1,),
                    mode=lax.GatherScatterMode.PROMISE_IN_BOUNDS)


def _mf_body(uid_hbm, iid_hbm, ue_hbm, ub_hbm, ie_hbm, ib_hbm, mean_hbm,
             out_hbm, uidx_v, iidx_v, urows_v, irows_v, ub_v, ib_v,
             dot_v, out_v, mean_v, sem, *, bpw):
  nc = 2
  wid = lax.axis_index("s") * nc + lax.axis_index("c")
  base = wid * bpw

  pltpu.sync_copy(uid_hbm.at[pl.ds(base, bpw)], uidx_v)
  pltpu.sync_copy(iid_hbm.at[pl.ds(base, bpw)], iidx_v)
  pltpu.sync_copy(mean_hbm, mean_v.at[pl.ds(0, 1)])

  cu = pltpu.async_copy(ue_hbm.at[uidx_v], urows_v, sem)
  ci = pltpu.async_copy(ie_hbm.at[iidx_v], irows_v, sem)
  cub = pltpu.async_copy(ub_hbm.at[uidx_v], ub_v, sem)
  cib = pltpu.async_copy(ib_hbm.at[iidx_v], ib_v, sem)
  cu.wait()
  ci.wait()
  cub.wait()
  cib.wait()

  nchunks = bpw // L

  lanes = lax.iota(jnp.int32, L)

  def chunk_body(c, _):
    r0 = c * L
    acc = jnp.zeros((L,), jnp.float32)
    for j in range(L):
      r = r0 + j
      u0 = urows_v[r, pl.ds(0, L)]
      u1 = urows_v[r, pl.ds(L, L)]
      i0 = irows_v[r, pl.ds(0, L)]
      i1 = irows_v[r, pl.ds(L, L)]
      s = u0 * i0 + u1 * i1
      for k in (8, 4, 2, 1):
        s = s + _shuffle(s, lanes ^ k)
      acc = jnp.where(lanes == j, s, acc)
    dot_v[pl.ds(r0, L)] = acc
    return _

  lax.fori_loop(0, nchunks, chunk_body, 0, unroll=False)

  m = mean_v[pl.ds(0, L)][0]

  def sig_body(c, _):
    sl = pl.ds(c * L, L)
    z = dot_v[sl] + ub_v[sl] + ib_v[sl] + m
    out_v[sl] = 1.0 / (1.0 + jnp.exp(-z))
    return _

  lax.fori_loop(0, nchunks, sig_body, 0, unroll=False)

  pltpu.sync_copy(out_v, out_hbm.at[pl.ds(base, bpw)])


@jax.jit
def kernel(x, user_emb, user_bias, item_emb, item_bias, mean):
  b = x.shape[0]
  nw = 32
  bpw = b // nw
  uid = x[:, 0]
  iid = x[:, 1]
  ubf = user_bias.reshape(-1)
  ibf = item_bias.reshape(-1)
  mesh = plsc.VectorSubcoreMesh(core_axis_name="c", subcore_axis_name="s")
  k = functools.partial(
      pl.kernel,
      mesh=mesh,
      compiler_params=pltpu.CompilerParams(use_tc_tiling_on_sc=False),
      out_type=jax.ShapeDtypeStruct((b,), jnp.float32),
      scratch_types=[
          pltpu.VMEM((bpw,), jnp.int32),
          pltpu.VMEM((bpw,), jnp.int32),
          pltpu.VMEM((bpw, D), jnp.float32),
          pltpu.VMEM((bpw, D), jnp.float32),
          pltpu.VMEM((bpw,), jnp.float32),
          pltpu.VMEM((bpw,), jnp.float32),
          pltpu.VMEM((bpw,), jnp.float32),
          pltpu.VMEM((bpw,), jnp.float32),
          pltpu.VMEM((L,), jnp.float32),
          pltpu.SemaphoreType.DMA,
      ],
  )(functools.partial(_mf_body, bpw=bpw))
  return k(uid, iid, user_emb, ubf, item_emb, ibf, mean)
